# Initial kernel scaffold; baseline (speedup 1.0000x reference)
import jax
import jax.numpy as jnp
from jax.experimental import pallas as pl


def kernel(inputs, u, logits):
    raise NotImplementedError("write your pallas kernel here")


# speedup vs baseline: 5.3450x; 5.3450x over previous
import jax
import jax.numpy as jnp
import numpy as np
from jax.experimental import pallas as pl

MUXI = 4096
MUXO = 64
_MININT = np.int32(-2147483648)


def _mask_body(u_ref, logit_ref, mask_ref):
    u = u_ref[...]
    logits = logit_ref[...]
    gn = -jnp.log(-jnp.log(u + 1e-20) + 1e-20)
    pert = logits + gn

    raw = jax.lax.bitcast_convert_type(pert, jnp.int32)
    key = raw ^ (jax.lax.shift_right_arithmetic(raw, 31) & jnp.int32(0x7FFFFFFF))

    bsz = u.shape[0]

    def bit_step(b, t_u):
        shift = 31 - b
        cand = t_u | jax.lax.shift_left(jnp.int32(1), shift)
        thr = cand ^ _MININT
        cnt = jnp.sum((key >= thr).astype(jnp.int32), axis=1, keepdims=True)
        return jnp.where(cnt >= MUXO, cand, t_u)

    t_u = jax.lax.fori_loop(0, 32, bit_step, jnp.zeros((bsz, 1), jnp.int32))
    thr = t_u ^ _MININT

    gt = key > thr
    eq = key == thr
    c1 = jnp.sum(gt.astype(jnp.int32), axis=1, keepdims=True)
    need = MUXO - c1
    idx = jax.lax.broadcasted_iota(jnp.int32, key.shape, 1)

    def bit_step2(b, sel_j):
        shift = 12 - b
        cand = sel_j | jax.lax.shift_left(jnp.int32(1), shift)
        cnt = jnp.sum((eq & (idx < cand)).astype(jnp.int32), axis=1,
                      keepdims=True)
        return jnp.where(cnt <= need, cand, sel_j)

    sel_j = jax.lax.fori_loop(0, 13, bit_step2, jnp.zeros((bsz, 1), jnp.int32))
    mask = gt | (eq & (idx < sel_j))
    mask_ref[...] = mask.astype(jnp.float32)


def _apply_body(x_ref, m_ref, o_ref):
    o_ref[...] = x_ref[...] * m_ref[...]


def kernel(inputs, u, logits):
    bsz = inputs.shape[0]
    u2 = u.reshape(bsz, MUXI)

    mask = pl.pallas_call(
        _mask_body,
        out_shape=jax.ShapeDtypeStruct((bsz, MUXI), jnp.float32),
    )(u2, logits)

    x = inputs.reshape(bsz, 64, MUXI)
    m3 = mask.reshape(bsz, 1, MUXI)
    out = pl.pallas_call(
        _apply_body,
        grid=(bsz,),
        in_specs=[
            pl.BlockSpec((1, 64, MUXI), lambda b: (b, 0, 0)),
            pl.BlockSpec((1, 1, MUXI), lambda b: (b, 0, 0)),
        ],
        out_specs=pl.BlockSpec((1, 64, MUXI), lambda b: (b, 0, 0)),
        out_shape=jax.ShapeDtypeStruct((bsz, 64, MUXI), jnp.float32),
    )(x, m3)
    return out.reshape(inputs.shape)
